# baseline (speedup 1.0000x reference)
import jax
import jax.numpy as jnp
from jax import lax
from jax.experimental import pallas as pl
from jax.experimental.pallas import tpu as pltpu
from jax.experimental.pallas import tpu_sc as plsc

DIM = 64
SCALE = 8.0
NC, NS = 2, 16
NW = NC * NS
NBATCH = 4096
SEQ = 200
WB = NBATCH // NW
TW = 128


def _emb_body(x_hbm, tab_hbm, z_hbm, xv, xT, g0, g1, zz0, zz1, semg, semo):
    wid = lax.axis_index("s") * NC + lax.axis_index("c")
    b0 = wid * WB
    pltpu.sync_copy(x_hbm.at[pl.ds(b0, WB)], xv)

    lanes = lax.iota(jnp.int32, 16)

    def xpose(t, c):
        tcol = jnp.full((16,), t, jnp.int32)
        for k in range(WB // 16):
            v = plsc.load_gather(xv, [lanes + (16 * k), tcol])
            xT[t, pl.ds(16 * k, 16)] = v
        return c

    lax.fori_loop(0, SEQ, xpose, 0)

    def fire_gather(t, gbuf):
        pltpu.make_async_copy(tab_hbm.at[xT.at[t]], gbuf, semg).start()

    def wait_gather(t, gbuf):
        pltpu.make_async_copy(tab_hbm.at[xT.at[t]], gbuf, semg).wait()

    def compute(gbuf, zzbuf):
        def per_d(d, c):
            dcol = jnp.full((16,), d, jnp.int32)
            for k in range(WB // 16):
                v = plsc.load_gather(gbuf, [lanes + (16 * k), dcol])
                zzbuf[d, pl.ds(16 * k, 16)] = v * SCALE
            return c

        lax.fori_loop(0, DIM, per_d, 0)

    def fire_out(t, zzbuf):
        for d8 in range(DIM // 8):
            pltpu.make_async_copy(
                zzbuf.at[pl.ds(d8 * 8, 8)], z_hbm.at[t, d8, wid], semo
            ).start()

    def wait_out(zzbuf):
        for d8 in range(DIM // 8):
            pltpu.make_async_copy(
                zzbuf.at[pl.ds(d8 * 8, 8)], z_hbm.at[0, d8, wid], semo
            ).wait()

    fire_gather(0, g0)

    def body2(i, c):
        t0 = 2 * i
        t1 = t0 + 1
        fire_gather(t1, g1)
        wait_gather(t0, g0)

        @pl.when(i >= 1)
        def _():
            wait_out(zz0)

        compute(g0, zz0)
        fire_out(t0, zz0)

        @pl.when(i <= (SEQ // 2 - 2))
        def _():
            fire_gather(t0 + 2, g0)

        wait_gather(t1, g1)

        @pl.when(i >= 1)
        def _():
            wait_out(zz1)

        compute(g1, zz1)
        fire_out(t1, zz1)
        return c

    lax.fori_loop(0, SEQ // 2, body2, 0)
    wait_out(zz0)
    wait_out(zz1)


@jax.jit
def kernel(x, table):
    tab_p = jnp.pad(table, ((0, 0), (0, TW - DIM)))
    mesh = plsc.VectorSubcoreMesh(core_axis_name="c", subcore_axis_name="s")
    z = pl.kernel(
        _emb_body,
        out_type=jax.ShapeDtypeStruct((SEQ, DIM // 8, NW, 8, 128), jnp.float32),
        mesh=mesh,
        compiler_params=pltpu.CompilerParams(
            use_tc_tiling_on_sc=False, needs_layout_passes=False
        ),
        scratch_types=[
            pltpu.VMEM((WB, SEQ), jnp.int32),
            pltpu.VMEM((SEQ, WB), jnp.int32),
            pltpu.VMEM((WB, TW), jnp.float32),
            pltpu.VMEM((WB, TW), jnp.float32),
            pltpu.VMEM((DIM, 128), jnp.float32),
            pltpu.VMEM((DIM, 128), jnp.float32),
            pltpu.SemaphoreType.DMA,
            pltpu.SemaphoreType.DMA,
        ],
    )(x.astype(jnp.int32), tab_p)
    return z.transpose(2, 4, 0, 1, 3).reshape(NBATCH, SEQ, DIM)


# speedup vs baseline: 1.6542x; 1.6542x over previous
import jax
import jax.numpy as jnp
from jax import lax
from jax.experimental import pallas as pl
from jax.experimental.pallas import tpu as pltpu
from jax.experimental.pallas import tpu_sc as plsc

DIM = 64
SCALE = 8.0
NC, NS = 2, 16
NW = NC * NS
NBATCH = 4096
SEQ = 200
WB = NBATCH // NW
TW = 128


def _emb_body(x_hbm, tab_hbm, z_hbm, xv, xT, g0, g1, zz0, zz1, semg, semo):
    wid = lax.axis_index("s") * NC + lax.axis_index("c")
    b0 = wid * WB
    pltpu.sync_copy(x_hbm.at[pl.ds(b0, WB)], xv)

    lanes = lax.iota(jnp.int32, 16)
    diag16 = [lax.rem(lanes + s, 16) for s in range(16)]
    diag8 = [lax.rem(lanes + s, 8) for s in range(8)]
    rows16 = [lanes + 16 * k for k in range(WB // 16)]

    def xpose(m, c):
        for k in range(WB // 16):
            for s in range(8):
                col = diag8[s] + 8 * m
                v = plsc.load_gather(xv, [rows16[k], col])
                plsc.store_scatter(xT, [col, rows16[k]], v)
        return c

    lax.fori_loop(0, SEQ // 8, xpose, 0)

    def fire_gather(t, gbuf):
        pltpu.make_async_copy(tab_hbm.at[xT.at[t]], gbuf, semg).start()

    def wait_gather(t, gbuf):
        pltpu.make_async_copy(tab_hbm.at[xT.at[t]], gbuf, semg).wait()

    def compute(gbuf, zzbuf):
        def per_k(k, c):
            row = lanes + 16 * k
            for j in range(DIM // 16):
                for s in range(16):
                    col = diag16[s] + 16 * j
                    v = plsc.load_gather(gbuf, [row, col])
                    plsc.store_scatter(zzbuf, [col, row], v * SCALE)
            return c

        lax.fori_loop(0, WB // 16, per_k, 0)

    def fire_out(t, zzbuf):
        for d8 in range(DIM // 8):
            pltpu.make_async_copy(
                zzbuf.at[pl.ds(d8 * 8, 8)], z_hbm.at[t, d8, wid], semo
            ).start()

    def wait_out(zzbuf):
        for d8 in range(DIM // 8):
            pltpu.make_async_copy(
                zzbuf.at[pl.ds(d8 * 8, 8)], z_hbm.at[0, d8, wid], semo
            ).wait()

    fire_gather(0, g0)

    def body2(i, c):
        t0 = 2 * i
        t1 = t0 + 1
        fire_gather(t1, g1)
        wait_gather(t0, g0)

        @pl.when(i >= 1)
        def _():
            wait_out(zz0)

        compute(g0, zz0)
        fire_out(t0, zz0)

        @pl.when(i <= (SEQ // 2 - 2))
        def _():
            fire_gather(t0 + 2, g0)

        wait_gather(t1, g1)

        @pl.when(i >= 1)
        def _():
            wait_out(zz1)

        compute(g1, zz1)
        fire_out(t1, zz1)
        return c

    lax.fori_loop(0, SEQ // 2, body2, 0)
    wait_out(zz0)
    wait_out(zz1)


@jax.jit
def kernel(x, table):
    tab_p = jnp.pad(table, ((0, 0), (0, TW - DIM)))
    mesh = plsc.VectorSubcoreMesh(core_axis_name="c", subcore_axis_name="s")
    z = pl.kernel(
        _emb_body,
        out_type=jax.ShapeDtypeStruct((SEQ, DIM // 8, NW, 8, 128), jnp.float32),
        mesh=mesh,
        compiler_params=pltpu.CompilerParams(
            use_tc_tiling_on_sc=False, needs_layout_passes=False
        ),
        scratch_types=[
            pltpu.VMEM((WB, SEQ), jnp.int32),
            pltpu.VMEM((SEQ, WB), jnp.int32),
            pltpu.VMEM((WB, TW), jnp.float32),
            pltpu.VMEM((WB, TW), jnp.float32),
            pltpu.VMEM((DIM, 128), jnp.float32),
            pltpu.VMEM((DIM, 128), jnp.float32),
            pltpu.SemaphoreType.DMA,
            pltpu.SemaphoreType.DMA,
        ],
    )(x.astype(jnp.int32), tab_p)
    return z.transpose(2, 4, 0, 1, 3).reshape(NBATCH, SEQ, DIM)


# speedup vs baseline: 1.8750x; 1.1335x over previous
import jax
import jax.numpy as jnp
from jax import lax
from jax.experimental import pallas as pl
from jax.experimental.pallas import tpu as pltpu
from jax.experimental.pallas import tpu_sc as plsc

DIM = 64
SCALE = 8.0
NC, NS = 2, 16
NW = NC * NS
NBATCH = 4096
SEQ = 200
WB = NBATCH // NW
TW = 128
SPLITS = ((0, 104), (104, 96))


def _emb_body(x_hbm, tab_hbm, z2_hbm, idx_all, r0, r1, semg, semo):
    wid = lax.axis_index("s") * NC + lax.axis_index("c")
    b0 = wid * WB
    flat0 = b0 * SEQ
    pltpu.sync_copy(x_hbm.at[pl.ds(b0, WB)], idx_all)

    def fire_gather(bi, rbuf):
        for off, ln in SPLITS:
            pltpu.make_async_copy(
                tab_hbm.at[idx_all.at[bi, pl.ds(off, ln)]],
                rbuf.at[pl.ds(off, ln)],
                semg,
            ).start()

    def wait_gather(bi, rbuf):
        for off, ln in SPLITS:
            pltpu.make_async_copy(
                tab_hbm.at[idx_all.at[bi, pl.ds(off, ln)]],
                rbuf.at[pl.ds(off, ln)],
                semg,
            ).wait()

    def compute(rbuf):
        def per_row(r, c):
            for c4 in range(DIM // 16):
                sl = pl.ds(c4 * 16, 16)
                rbuf[r, sl] = rbuf[r, sl] * SCALE
            return c

        lax.fori_loop(0, SEQ, per_row, 0)

    def fire_out(bi, rbuf):
        pltpu.make_async_copy(
            rbuf, z2_hbm.at[pl.ds(flat0 + bi * SEQ, SEQ)], semo
        ).start()

    def wait_out(rbuf):
        pltpu.make_async_copy(
            rbuf, z2_hbm.at[pl.ds(flat0, SEQ)], semo
        ).wait()

    fire_gather(0, r0)

    def body2(i, c):
        bi0 = 2 * i
        bi1 = bi0 + 1
        fire_gather(bi1, r1)
        wait_gather(bi0, r0)

        @pl.when(i >= 1)
        def _():
            wait_out(r0)

        compute(r0)
        fire_out(bi0, r0)

        @pl.when(i <= (WB // 2 - 2))
        def _():
            fire_gather(bi0 + 2, r0)

        wait_gather(bi1, r1)

        @pl.when(i >= 1)
        def _():
            wait_out(r1)

        compute(r1)
        fire_out(bi1, r1)
        return c

    lax.fori_loop(0, WB // 2, body2, 0)
    wait_out(r0)
    wait_out(r1)


@jax.jit
def kernel(x, table):
    tab_p = jnp.pad(table, ((0, 0), (0, TW - DIM)))
    mesh = plsc.VectorSubcoreMesh(core_axis_name="c", subcore_axis_name="s")
    z2 = pl.kernel(
        _emb_body,
        out_type=jax.ShapeDtypeStruct((NBATCH * SEQ, TW), jnp.float32),
        mesh=mesh,
        compiler_params=pltpu.CompilerParams(
            use_tc_tiling_on_sc=False, needs_layout_passes=False
        ),
        scratch_types=[
            pltpu.VMEM((WB, SEQ), jnp.int32),
            pltpu.VMEM((SEQ, TW), jnp.float32),
            pltpu.VMEM((SEQ, TW), jnp.float32),
            pltpu.SemaphoreType.DMA,
            pltpu.SemaphoreType.DMA,
        ],
    )(x.astype(jnp.int32), tab_p)
    return z2[:, :DIM].reshape(NBATCH, SEQ, DIM)
